# baseline (speedup 1.0000x reference)
import functools

import jax
import jax.numpy as jnp
from jax import lax
from jax.experimental import pallas as pl
from jax.experimental.pallas import tpu as pltpu
from jax.experimental.pallas import tpu_sc as plsc

EPS = 1e-5

NUM_CORES = 2
NUM_SUBCORES = 16
NUM_WORKERS = NUM_CORES * NUM_SUBCORES

GROUP = 512
GROUPS_PER_SUPER = 1
SUPER = GROUP * GROUPS_PER_SUPER


def _sc_gather_paired(ids2d, table, n_tok, d_emb):
    n_per_w = n_tok // NUM_WORKERS
    n_groups = n_per_w // GROUP
    n_super = n_per_w // SUPER
    half_rows = n_tok // 2
    half_workers = NUM_WORKERS // 2

    mesh = plsc.VectorSubcoreMesh(core_axis_name="c", subcore_axis_name="s")

    @functools.partial(
        pl.kernel,
        mesh=mesh,
        out_type=jax.ShapeDtypeStruct((half_rows, 2 * d_emb), jnp.float32),
        scratch_types=[
            pltpu.VMEM((n_groups, GROUP), jnp.int32),
            pltpu.VMEM((3, SUPER, d_emb), jnp.float32),
            pltpu.SemaphoreType.DMA((3,)),
            pltpu.SemaphoreType.DMA((3,)),
        ],
        compiler_params=pltpu.CompilerParams(use_tc_tiling_on_sc=False),
    )
    def gather_kernel(ids_hbm, table_hbm, out_hbm, idx_v, rows_v, gsem, ssem):
        wid = lax.axis_index("s") * NUM_CORES + lax.axis_index("c")
        half = wid // half_workers
        row_base = (wid % half_workers) * n_per_w
        col = half * d_emb

        pltpu.sync_copy(ids_hbm.at[wid], idx_v)

        def fire(s):
            buf = lax.rem(s, 3)
            for b in range(GROUPS_PER_SUPER):
                pltpu.async_copy(
                    table_hbm.at[idx_v.at[s * GROUPS_PER_SUPER + b]],
                    rows_v.at[buf, pl.ds(b * GROUP, GROUP)],
                    gsem.at[buf],
                )

        def drain_gather(s):
            buf = lax.rem(s, 3)
            for b in range(GROUPS_PER_SUPER):
                pltpu.make_async_copy(
                    table_hbm.at[idx_v.at[s * GROUPS_PER_SUPER + b]],
                    rows_v.at[buf, pl.ds(b * GROUP, GROUP)],
                    gsem.at[buf],
                ).wait()

        def out_slice(s):
            return out_hbm.at[
                pl.ds(row_base + s * SUPER, SUPER), pl.ds(col, d_emb)]

        def store(s):
            buf = lax.rem(s, 3)
            pltpu.async_copy(rows_v.at[buf], out_slice(s), ssem.at[buf])

        def drain_store(s):
            buf = lax.rem(s, 3)
            pltpu.make_async_copy(rows_v.at[buf], out_slice(s), ssem.at[buf]).wait()

        fire(0)
        fire(1)

        def body(s, carry):
            drain_gather(s)
            store(s)

            @pl.when(s >= 1)
            def _():
                drain_store(s - 1)

            @pl.when(s + 2 < n_super)
            def _():
                fire(s + 2)

            return carry

        lax.fori_loop(0, n_super, body, 0)
        drain_store(n_super - 1)

    return gather_kernel(ids2d, table)


def _tc_proj_ln(emb2, w2, gamma, beta, half_rows, d_model, blk):
    def body(emb_ref, w_ref, g_ref, b_ref, out_ref):
        e2 = emb_ref[...]
        proj = lax.dot_general(
            e2, w_ref[...], (((1,), (0,)), ((), ())),
            preferred_element_type=jnp.float32)
        g = g_ref[...]
        b = b_ref[...]
        for h in range(2):
            p = proj[:, h * d_model:(h + 1) * d_model]
            mu = jnp.mean(p, axis=1, keepdims=True)
            diff = p - mu
            var = jnp.mean(diff * diff, axis=1, keepdims=True)
            inv = lax.rsqrt(var + EPS)
            out_ref[h] = diff * inv * g + b

    grid = (half_rows // blk,)
    return pl.pallas_call(
        body,
        grid=grid,
        in_specs=[
            pl.BlockSpec((blk, 128), lambda i: (i, 0)),
            pl.BlockSpec((128, 2 * d_model), lambda i: (0, 0)),
            pl.BlockSpec((1, d_model), lambda i: (0, 0)),
            pl.BlockSpec((1, d_model), lambda i: (0, 0)),
        ],
        out_specs=pl.BlockSpec((2, blk, d_model), lambda i: (0, i, 0)),
        out_shape=jax.ShapeDtypeStruct((2, half_rows, d_model), jnp.float32),
    )(emb2, w2, gamma, beta)


def kernel(token_ids, table, W, gamma, beta):
    b, h = token_ids.shape
    vocab, d_emb = table.shape
    d_model = W.shape[0]
    n_tok = b * h
    half_rows = n_tok // 2

    n_per_w = n_tok // NUM_WORKERS
    ids2d = token_ids.reshape(NUM_WORKERS, n_per_w // GROUP, GROUP).astype(jnp.int32)

    emb2 = _sc_gather_paired(ids2d, table, n_tok, d_emb)

    wt = W.T
    zeros = jnp.zeros((d_emb, d_model), jnp.float32)
    w2 = jnp.concatenate([
        jnp.concatenate([wt, zeros], axis=1),
        jnp.concatenate([zeros, wt], axis=1),
    ], axis=0)

    out = _tc_proj_ln(
        emb2, w2, gamma.reshape(1, d_model), beta.reshape(1, d_model),
        half_rows, d_model, blk=2048)
    return out.reshape(b, h, d_model)


# speedup vs baseline: 1.0253x; 1.0253x over previous
import functools

import jax
import jax.numpy as jnp
from jax import lax
from jax.experimental import pallas as pl
from jax.experimental.pallas import tpu as pltpu
from jax.experimental.pallas import tpu_sc as plsc

EPS = 1e-5

NUM_CORES = 2
NUM_SUBCORES = 16
NUM_WORKERS = NUM_CORES * NUM_SUBCORES

K_CHUNKS = 2
GROUP = 512


def _sc_gather_paired(ids3d, table, n_tok, d_emb):
    n_per_w = n_tok // NUM_WORKERS
    n_super = n_per_w // GROUP
    half_rows = n_tok // 2
    half_workers = NUM_WORKERS // 2

    mesh = plsc.VectorSubcoreMesh(core_axis_name="c", subcore_axis_name="s")

    @functools.partial(
        pl.kernel,
        mesh=mesh,
        out_type=jax.ShapeDtypeStruct((half_rows, 2 * d_emb), jnp.float32),
        scratch_types=[
            pltpu.VMEM((n_super, GROUP), jnp.int32),
            pltpu.VMEM((3, GROUP, d_emb), jnp.float32),
            pltpu.SemaphoreType.DMA((3,)),
            pltpu.SemaphoreType.DMA((3,)),
        ],
        compiler_params=pltpu.CompilerParams(use_tc_tiling_on_sc=False),
    )
    def gather_kernel(ids_hbm, table_hbm, out_hbm, idx_v, rows_v, gsem, ssem):
        wid = lax.axis_index("s") * NUM_CORES + lax.axis_index("c")
        half = wid // half_workers
        row_base = (wid % half_workers) * n_per_w
        col = half * d_emb

        pltpu.sync_copy(ids_hbm.at[wid], idx_v)

        def gather_copy(s):
            buf = lax.rem(s, 3)
            return pltpu.make_async_copy(
                table_hbm.at[idx_v.at[s]], rows_v.at[buf], gsem.at[buf])

        def store_copy(s):
            buf = lax.rem(s, 3)
            dst = out_hbm.at[
                pl.ds(row_base + s * GROUP, GROUP), pl.ds(col, d_emb)]
            return pltpu.make_async_copy(rows_v.at[buf], dst, ssem.at[buf])

        gather_copy(0).start()
        gather_copy(1).start()

        def body(s, carry):
            gather_copy(s).wait()
            store_copy(s).start()

            @pl.when(s >= 1)
            def _():
                store_copy(s - 1).wait()

            @pl.when(s + 2 < n_super)
            def _():
                gather_copy(s + 2).start()

            return carry

        lax.fori_loop(0, n_super, body, 0)
        store_copy(n_super - 1).wait()

    return gather_kernel(ids3d, table)


def _tc_proj_ln_chunk(emb2, w2, gamma, beta, out_prev, chunk, k, half_rows,
                      d_model, blk):
    def body(*refs):
        emb_ref, w_ref, g_ref, b_ref = refs[:4]
        out_ref = refs[-1]
        e2 = emb_ref[...]
        proj = lax.dot_general(
            e2, w_ref[...], (((1,), (0,)), ((), ())),
            preferred_element_type=jnp.float32)
        g = g_ref[...]
        b = b_ref[...]
        for h in range(2):
            p = proj[:, h * d_model:(h + 1) * d_model]
            mu = jnp.mean(p, axis=1, keepdims=True)
            diff = p - mu
            var = jnp.mean(diff * diff, axis=1, keepdims=True)
            inv = lax.rsqrt(var + EPS)
            out_ref[0, h] = diff * inv * g + b

    in_specs = [
        pl.BlockSpec((blk, 128), lambda i: (i, 0)),
        pl.BlockSpec((128, 2 * d_model), lambda i: (0, 0)),
        pl.BlockSpec((1, d_model), lambda i: (0, 0)),
        pl.BlockSpec((1, d_model), lambda i: (0, 0)),
    ]
    inputs = [emb2, w2, gamma, beta]
    aliases = {}
    if out_prev is not None:
        in_specs.append(pl.BlockSpec(memory_space=pl.ANY))
        inputs.append(out_prev)
        aliases = {4: 0}

    return pl.pallas_call(
        body,
        grid=(half_rows // blk,),
        in_specs=in_specs,
        out_specs=pl.BlockSpec(
            (1, 2, blk, d_model), lambda i, c=chunk: (c, 0, i, 0)),
        out_shape=jax.ShapeDtypeStruct((k, 2, half_rows, d_model),
                                       jnp.float32),
        input_output_aliases=aliases,
    )(*inputs)


def kernel(token_ids, table, W, gamma, beta):
    b, h = token_ids.shape
    vocab, d_emb = table.shape
    d_model = W.shape[0]
    n_tok = b * h
    k = K_CHUNKS
    t_chunk = n_tok // k
    half_rows = t_chunk // 2
    n_per_w = t_chunk // NUM_WORKERS

    ids4d = token_ids.reshape(
        k, NUM_WORKERS, n_per_w // GROUP, GROUP).astype(jnp.int32)

    wt = W.T
    zeros = jnp.zeros((d_emb, d_model), jnp.float32)
    w2 = jnp.concatenate([
        jnp.concatenate([wt, zeros], axis=1),
        jnp.concatenate([zeros, wt], axis=1),
    ], axis=0)
    g2 = gamma.reshape(1, d_model)
    b2 = beta.reshape(1, d_model)

    out = None
    for c in range(k):
        emb2 = _sc_gather_paired(ids4d[c], table, t_chunk, d_emb)
        out = _tc_proj_ln_chunk(
            emb2, w2, g2, b2, out, c, k, half_rows, d_model, blk=2048)
    return out.reshape(b, h, d_model)


# speedup vs baseline: 1.2144x; 1.1845x over previous
import functools

import jax
import jax.numpy as jnp
from jax import lax
from jax.experimental import pallas as pl
from jax.experimental.pallas import tpu as pltpu
from jax.experimental.pallas import tpu_sc as plsc

EPS = 1e-5

NUM_CORES = 2
NUM_SUBCORES = 16
NUM_WORKERS = NUM_CORES * NUM_SUBCORES

GROUP = 256


def _tc_table_proj_ln(tT, w, gamma, beta, vocab, d_emb, d_model, blk):
    def body(t_ref, w_ref, g_ref, b_ref, out_ref):
        t = t_ref[...]
        proj = lax.dot_general(
            t, w_ref[...], (((0,), (1,)), ((), ())),
            preferred_element_type=jnp.float32)
        mu = jnp.mean(proj, axis=1, keepdims=True)
        diff = proj - mu
        var = jnp.mean(diff * diff, axis=1, keepdims=True)
        inv = lax.rsqrt(var + EPS)
        out_ref[...] = diff * inv * g_ref[...] + b_ref[...]

    grid = (pl.cdiv(vocab, blk),)
    return pl.pallas_call(
        body,
        grid=grid,
        in_specs=[
            pl.BlockSpec((d_emb, blk), lambda i: (0, i)),
            pl.BlockSpec((d_model, d_emb), lambda i: (0, 0)),
            pl.BlockSpec((1, d_model), lambda i: (0, 0)),
            pl.BlockSpec((1, d_model), lambda i: (0, 0)),
        ],
        out_specs=pl.BlockSpec((blk, d_model), lambda i: (i, 0)),
        out_shape=jax.ShapeDtypeStruct((vocab, d_model), jnp.float32),
    )(tT, w, gamma, beta)


def _sc_gather_rows(ids3d, rows_hbm_src, n_tok, d_model):
    n_per_w = n_tok // NUM_WORKERS
    n_super = n_per_w // GROUP

    mesh = plsc.VectorSubcoreMesh(core_axis_name="c", subcore_axis_name="s")

    @functools.partial(
        pl.kernel,
        mesh=mesh,
        out_type=jax.ShapeDtypeStruct((n_tok, d_model), jnp.float32),
        scratch_types=[
            pltpu.VMEM((n_super, GROUP), jnp.int32),
            pltpu.VMEM((3, GROUP, d_model), jnp.float32),
            pltpu.SemaphoreType.DMA((3,)),
            pltpu.SemaphoreType.DMA((3,)),
        ],
        compiler_params=pltpu.CompilerParams(use_tc_tiling_on_sc=False),
    )
    def gather_kernel(ids_hbm, table_hbm, out_hbm, idx_v, rows_v, gsem, ssem):
        wid = lax.axis_index("s") * NUM_CORES + lax.axis_index("c")
        row_base = wid * n_per_w

        pltpu.sync_copy(ids_hbm.at[wid], idx_v)

        def gather_copy(s):
            buf = lax.rem(s, 3)
            return pltpu.make_async_copy(
                table_hbm.at[idx_v.at[s]], rows_v.at[buf], gsem.at[buf])

        def store_copy(s):
            buf = lax.rem(s, 3)
            dst = out_hbm.at[pl.ds(row_base + s * GROUP, GROUP)]
            return pltpu.make_async_copy(rows_v.at[buf], dst, ssem.at[buf])

        gather_copy(0).start()
        gather_copy(1).start()

        def body(s, carry):
            gather_copy(s).wait()
            store_copy(s).start()

            @pl.when(s >= 1)
            def _():
                store_copy(s - 1).wait()

            @pl.when(s + 2 < n_super)
            def _():
                gather_copy(s + 2).start()

            return carry

        lax.fori_loop(0, n_super, body, 0)
        store_copy(n_super - 1).wait()

    return gather_kernel(ids3d, rows_hbm_src)


def kernel(token_ids, table, W, gamma, beta):
    b, h = token_ids.shape
    vocab, d_emb = table.shape
    d_model = W.shape[0]
    n_tok = b * h
    n_per_w = n_tok // NUM_WORKERS

    ids3d = token_ids.reshape(
        NUM_WORKERS, n_per_w // GROUP, GROUP).astype(jnp.int32)

    out_table = _tc_table_proj_ln(
        table.T, W, gamma.reshape(1, d_model), beta.reshape(1, d_model),
        vocab, d_emb, d_model, blk=2048)
    out = _sc_gather_rows(ids3d, out_table, n_tok, d_model)
    return out.reshape(b, h, d_model)


# speedup vs baseline: 1.4834x; 1.2215x over previous
import functools

import jax
import jax.numpy as jnp
from jax import lax
from jax.experimental import pallas as pl
from jax.experimental.pallas import tpu as pltpu
from jax.experimental.pallas import tpu_sc as plsc

EPS = 1e-5

NUM_CORES = 2
NUM_SUBCORES = 16
NUM_WORKERS = NUM_CORES * NUM_SUBCORES

GROUP = 256


def _tc_table_proj_ln(tT, w, gamma, beta, vocab, d_emb, d_model, blk):
    def body(t_ref, w_ref, g_ref, b_ref, out_ref):
        t = t_ref[...]
        proj = lax.dot_general(
            t, w_ref[...], (((0,), (1,)), ((), ())),
            preferred_element_type=jnp.float32)
        mu = jnp.mean(proj, axis=1, keepdims=True)
        diff = proj - mu
        var = jnp.mean(diff * diff, axis=1, keepdims=True)
        inv = lax.rsqrt(var + EPS)
        out_ref[...] = diff * inv * g_ref[...] + b_ref[...]

    grid = (pl.cdiv(vocab, blk),)
    return pl.pallas_call(
        body,
        grid=grid,
        in_specs=[
            pl.BlockSpec((d_emb, blk), lambda i: (0, i)),
            pl.BlockSpec((d_model, d_emb), lambda i: (0, 0)),
            pl.BlockSpec((1, d_model), lambda i: (0, 0)),
            pl.BlockSpec((1, d_model), lambda i: (0, 0)),
        ],
        out_specs=pl.BlockSpec((blk, d_model), lambda i: (i, 0)),
        out_shape=jax.ShapeDtypeStruct((vocab, d_model), jnp.float32),
    )(tT, w, gamma, beta)


def _sc_gather_rows(ids3d, rows_hbm_src, n_tok, d_model):
    n_per_w = n_tok // NUM_WORKERS
    n_super = n_per_w // GROUP

    mesh = plsc.VectorSubcoreMesh(core_axis_name="c", subcore_axis_name="s")

    @functools.partial(
        pl.kernel,
        mesh=mesh,
        out_type=jax.ShapeDtypeStruct((n_tok, d_model), jnp.float32),
        scratch_types=[
            pltpu.VMEM((n_super, GROUP), jnp.int32),
            pltpu.VMEM((3, GROUP, d_model), jnp.float32),
            pltpu.SemaphoreType.DMA((3,)),
            pltpu.SemaphoreType.DMA((3,)),
        ],
        compiler_params=pltpu.CompilerParams(use_tc_tiling_on_sc=False),
    )
    def gather_kernel(ids_hbm, table_hbm, out_hbm, idx_v, rows_v, gsem, ssem):
        wid = lax.axis_index("s") * NUM_CORES + lax.axis_index("c")
        row_base = wid * n_per_w

        pltpu.sync_copy(ids_hbm.at[wid], idx_v)

        def gather_copy(s):
            buf = lax.rem(s, 3)
            return pltpu.make_async_copy(
                table_hbm.at[idx_v.at[s]], rows_v.at[buf], gsem.at[buf])

        def store_copy(s):
            buf = lax.rem(s, 3)
            dst = out_hbm.at[pl.ds(row_base + s * GROUP, GROUP)]
            return pltpu.make_async_copy(rows_v.at[buf], dst, ssem.at[buf])

        gather_copy(0).start()
        gather_copy(1).start()

        def body(s, carry):
            gather_copy(s).wait()
            store_copy(s).start()

            @pl.when(s >= 1)
            def _():
                store_copy(s - 1).wait()

            @pl.when(s + 2 < n_super)
            def _():
                gather_copy(s + 2).start()

            return carry

        lax.fori_loop(0, n_super, body, 0)
        store_copy(n_super - 1).wait()

    return gather_kernel(ids3d, rows_hbm_src)


def kernel(token_ids, table, W, gamma, beta):
    b, h = token_ids.shape
    vocab, d_emb = table.shape
    d_model = W.shape[0]
    n_tok = b * h
    n_per_w = n_tok // NUM_WORKERS

    ids3d = token_ids.reshape(
        NUM_WORKERS, n_per_w // GROUP, GROUP).astype(jnp.int32)

    out_table = _tc_table_proj_ln(
        table.T, W, gamma.reshape(1, d_model), beta.reshape(1, d_model),
        vocab, d_emb, d_model, blk=8192)
    out = _sc_gather_rows(ids3d, out_table, n_tok, d_model)
    return out.reshape(b, h, d_model)


# speedup vs baseline: 1.4919x; 1.0057x over previous
import functools

import jax
import jax.numpy as jnp
from jax import lax
from jax.experimental import pallas as pl
from jax.experimental.pallas import tpu as pltpu
from jax.experimental.pallas import tpu_sc as plsc

EPS = 1e-5

NUM_CORES = 2
NUM_SUBCORES = 16
NUM_WORKERS = NUM_CORES * NUM_SUBCORES

GROUP = 256


def _tc_table_proj_ln(tT, w, gamma, beta, vocab, d_emb, d_model, blk):
    def body(t_ref, w_ref, g_ref, b_ref, out_ref):
        t = t_ref[...]
        proj = lax.dot_general(
            t, w_ref[...], (((0,), (1,)), ((), ())),
            preferred_element_type=jnp.float32)
        mu = jnp.mean(proj, axis=1, keepdims=True)
        diff = proj - mu
        var = jnp.mean(diff * diff, axis=1, keepdims=True)
        inv = lax.rsqrt(var + EPS)
        out_ref[...] = diff * inv * g_ref[...] + b_ref[...]

    grid = (pl.cdiv(vocab, blk),)
    return pl.pallas_call(
        body,
        grid=grid,
        in_specs=[
            pl.BlockSpec((d_emb, blk), lambda i: (0, i)),
            pl.BlockSpec((d_model, d_emb), lambda i: (0, 0)),
            pl.BlockSpec((1, d_model), lambda i: (0, 0)),
            pl.BlockSpec((1, d_model), lambda i: (0, 0)),
        ],
        out_specs=pl.BlockSpec((blk, d_model), lambda i: (i, 0)),
        out_shape=jax.ShapeDtypeStruct((vocab, d_model), jnp.float32),
    )(tT, w, gamma, beta)


def _sc_gather_rows(ids3d, rows_hbm_src, n_tok, d_model):
    n_per_w = n_tok // NUM_WORKERS
    n_super = n_per_w // GROUP

    mesh = plsc.VectorSubcoreMesh(core_axis_name="c", subcore_axis_name="s")

    @functools.partial(
        pl.kernel,
        mesh=mesh,
        out_type=jax.ShapeDtypeStruct((n_tok, d_model), jnp.float32),
        scratch_types=[
            pltpu.VMEM((n_super, GROUP), jnp.int32),
            pltpu.VMEM((3, GROUP, d_model), jnp.float32),
            pltpu.SemaphoreType.DMA((3,)),
            pltpu.SemaphoreType.DMA((3,)),
        ],
        compiler_params=pltpu.CompilerParams(use_tc_tiling_on_sc=False),
    )
    def gather_kernel(ids_hbm, table_hbm, out_hbm, idx_v, rows_v, gsem, ssem):
        wid = lax.axis_index("s") * NUM_CORES + lax.axis_index("c")
        row_base = wid * n_per_w

        pltpu.sync_copy(ids_hbm.at[wid], idx_v)

        def gather_copy(s):
            buf = lax.rem(s, 3)
            return pltpu.make_async_copy(
                table_hbm.at[idx_v.at[s]], rows_v.at[buf], gsem.at[buf])

        def store_copy(s):
            buf = lax.rem(s, 3)
            dst = out_hbm.at[pl.ds(row_base + s * GROUP, GROUP)]
            return pltpu.make_async_copy(rows_v.at[buf], dst, ssem.at[buf])

        gather_copy(0).start()
        gather_copy(1).start()

        def body(s, carry):
            gather_copy(s).wait()
            store_copy(s).start()

            @pl.when(s >= 1)
            def _():
                store_copy(s - 1).wait()

            @pl.when(s + 2 < n_super)
            def _():
                gather_copy(s + 2).start()

            return carry

        lax.fori_loop(0, n_super, body, 0)
        store_copy(n_super - 1).wait()

    return gather_kernel(ids3d, rows_hbm_src)


def kernel(token_ids, table, W, gamma, beta):
    b, h = token_ids.shape
    vocab, d_emb = table.shape
    d_model = W.shape[0]
    n_tok = b * h
    n_per_w = n_tok // NUM_WORKERS

    ids3d = token_ids.reshape(
        NUM_WORKERS, n_per_w // GROUP, GROUP).astype(jnp.int32)

    out_table = _tc_table_proj_ln(
        table.T, W, gamma.reshape(1, d_model), beta.reshape(1, d_model),
        vocab, d_emb, d_model, blk=16384)
    out = _sc_gather_rows(ids3d, out_table, n_tok, d_model)
    return out.reshape(b, h, d_model)


# speedup vs baseline: 1.4926x; 1.0005x over previous
import functools

import jax
import jax.numpy as jnp
from jax import lax
from jax.experimental import pallas as pl
from jax.experimental.pallas import tpu as pltpu
from jax.experimental.pallas import tpu_sc as plsc

EPS = 1e-5

NUM_CORES = 2
NUM_SUBCORES = 16
NUM_WORKERS = NUM_CORES * NUM_SUBCORES

GROUP = 128


def _tc_table_proj_ln(tT, w, gamma, beta, vocab, d_emb, d_model, blk):
    def body(t_ref, w_ref, g_ref, b_ref, out_ref):
        t = t_ref[...]
        proj = lax.dot_general(
            t, w_ref[...], (((0,), (1,)), ((), ())),
            preferred_element_type=jnp.float32)
        mu = jnp.mean(proj, axis=1, keepdims=True)
        diff = proj - mu
        var = jnp.mean(diff * diff, axis=1, keepdims=True)
        inv = lax.rsqrt(var + EPS)
        out_ref[...] = diff * inv * g_ref[...] + b_ref[...]

    grid = (pl.cdiv(vocab, blk),)
    return pl.pallas_call(
        body,
        grid=grid,
        in_specs=[
            pl.BlockSpec((d_emb, blk), lambda i: (0, i)),
            pl.BlockSpec((d_model, d_emb), lambda i: (0, 0)),
            pl.BlockSpec((1, d_model), lambda i: (0, 0)),
            pl.BlockSpec((1, d_model), lambda i: (0, 0)),
        ],
        out_specs=pl.BlockSpec((blk, d_model), lambda i: (i, 0)),
        out_shape=jax.ShapeDtypeStruct((vocab, d_model), jnp.float32),
    )(tT, w, gamma, beta)


def _sc_gather_rows(ids3d, rows_hbm_src, n_tok, d_model):
    n_per_w = n_tok // NUM_WORKERS
    n_super = n_per_w // GROUP

    mesh = plsc.VectorSubcoreMesh(core_axis_name="c", subcore_axis_name="s")

    @functools.partial(
        pl.kernel,
        mesh=mesh,
        out_type=jax.ShapeDtypeStruct((n_tok, d_model), jnp.float32),
        scratch_types=[
            pltpu.VMEM((n_super, GROUP), jnp.int32),
            pltpu.VMEM((4, GROUP, d_model), jnp.float32),
            pltpu.SemaphoreType.DMA((4,)),
            pltpu.SemaphoreType.DMA((4,)),
        ],
        compiler_params=pltpu.CompilerParams(use_tc_tiling_on_sc=False),
    )
    def gather_kernel(ids_hbm, table_hbm, out_hbm, idx_v, rows_v, gsem, ssem):
        wid = lax.axis_index("s") * NUM_CORES + lax.axis_index("c")
        row_base = wid * n_per_w

        pltpu.sync_copy(ids_hbm.at[wid], idx_v)

        def gather_copy(s):
            buf = lax.rem(s, 4)
            return pltpu.make_async_copy(
                table_hbm.at[idx_v.at[s]], rows_v.at[buf], gsem.at[buf])

        def store_copy(s):
            buf = lax.rem(s, 4)
            dst = out_hbm.at[pl.ds(row_base + s * GROUP, GROUP)]
            return pltpu.make_async_copy(rows_v.at[buf], dst, ssem.at[buf])

        gather_copy(0).start()
        gather_copy(1).start()
        gather_copy(2).start()

        def body(s, carry):
            gather_copy(s).wait()
            store_copy(s).start()

            @pl.when(s >= 1)
            def _():
                store_copy(s - 1).wait()

            @pl.when(s + 3 < n_super)
            def _():
                gather_copy(s + 3).start()

            return carry

        lax.fori_loop(0, n_super, body, 0)
        store_copy(n_super - 1).wait()

    return gather_kernel(ids3d, rows_hbm_src)


def kernel(token_ids, table, W, gamma, beta):
    b, h = token_ids.shape
    vocab, d_emb = table.shape
    d_model = W.shape[0]
    n_tok = b * h
    n_per_w = n_tok // NUM_WORKERS

    ids3d = token_ids.reshape(
        NUM_WORKERS, n_per_w // GROUP, GROUP).astype(jnp.int32)

    out_table = _tc_table_proj_ln(
        table.T, W, gamma.reshape(1, d_model), beta.reshape(1, d_model),
        vocab, d_emb, d_model, blk=16384)
    out = _sc_gather_rows(ids3d, out_table, n_tok, d_model)
    return out.reshape(b, h, d_model)
